# Initial kernel scaffold; baseline (speedup 1.0000x reference)
import jax
import jax.numpy as jnp
from jax.experimental import pallas as pl


def kernel(x, edge_index, edge_attr, u, We, be, Wn, bn):
    raise NotImplementedError("write your pallas kernel here")


# speedup vs baseline: 3.4403x; 3.4403x over previous
import functools

import jax
import jax.numpy as jnp
from jax import lax
from jax.experimental import pallas as pl
from jax.experimental.pallas import tpu as pltpu, tpu_sc as plsc

N = 10000
E = 320000
DF = 128
DE = 16
DG = 32
DEH = 128

NC = 2
NS = 16
NW = NC * NS
EW = E // NW
C = 80
NCHUNK = EW // C
ZCH = (N // C + NS - 1) // NS


def _pre_body(x_ref, ws_ref, wd_ref, u_ref, weu_ref, wnu_ref, be_ref, bn_ref,
              xs_ref, xd_ref, ce_ref, cn_ref):
    x = x_ref[...]
    xs_ref[...] = jnp.dot(x, ws_ref[...], preferred_element_type=jnp.float32)
    xd_ref[...] = jnp.dot(x, wd_ref[...], preferred_element_type=jnp.float32)
    u = u_ref[...]
    ce_ref[...] = jnp.dot(u, weu_ref[...], preferred_element_type=jnp.float32) + be_ref[...]
    cn_ref[...] = jnp.dot(u, wnu_ref[...], preferred_element_type=jnp.float32) + bn_ref[...]


def _ea_body(attr_ref, wea_ref, ce_ref, ea_ref):
    ea_ref[...] = (
        jnp.dot(attr_ref[...], wea_ref[...], preferred_element_type=jnp.float32)
        + ce_ref[...]
    )


def _node_body(x_ref, agg_ref, wnx_ref, wna_ref, cn_ref, out_ref):
    agg = agg_ref[0:N, :] + agg_ref[N:2 * N, :]
    acc = jnp.dot(x_ref[...], wnx_ref[...], preferred_element_type=jnp.float32)
    acc += jnp.dot(agg, wna_ref[...], preferred_element_type=jnp.float32)
    out_ref[...] = jnp.maximum(acc + cn_ref[...], 0.0)


def _sc_edge_body(ea_hbm, src_hbm, dst_hbm, xs_hbm, xd_hbm, out_hbm,
                  srcbuf, dstbuf, m, gs, gd, agg_sh, sem1, sem2):
    cid = lax.axis_index("c")
    sid = lax.axis_index("s")
    wid = cid * NS + sid
    ebase = wid * EW

    zero = jnp.zeros((16,), jnp.float32)

    def zero_m(r, _):
        for j in range(DEH // 16):
            m[r, pl.ds(j * 16, 16)] = zero
        return 0

    lax.fori_loop(0, C, zero_m, 0)

    def zero_agg(i, _):
        k = sid * ZCH + i

        @pl.when(k < N // C)
        def _():
            pltpu.sync_copy(m, agg_sh.at[pl.ds(k * C, C)])
        return 0

    lax.fori_loop(0, ZCH, zero_agg, 0)
    plsc.subcore_barrier()

    def step(i, _):
        off = ebase + i * C
        pltpu.sync_copy(src_hbm.at[pl.ds(off, C)], srcbuf)
        pltpu.sync_copy(dst_hbm.at[pl.ds(off, C)], dstbuf)
        cp1 = pltpu.async_copy(xs_hbm.at[srcbuf], gs, sem1)
        cp2 = pltpu.async_copy(xd_hbm.at[dstbuf], gd, sem2)
        pltpu.sync_copy(ea_hbm.at[pl.ds(off, C)], m)
        cp1.wait()
        cp2.wait()

        def relu_row(r, _):
            for j in range(DEH // 16):
                s = pl.ds(j * 16, 16)
                v = m[r, s] + gs[r, s] + gd[r, s]
                m[r, s] = jnp.maximum(v, 0.0)
            return 0

        lax.fori_loop(0, C, relu_row, 0)
        pltpu.sync_copy(m, agg_sh.at[dstbuf], add=True)
        return 0

    lax.fori_loop(0, NCHUNK, step, 0)
    plsc.subcore_barrier()

    def wb(i, _):
        k = sid * ZCH + i

        @pl.when(k < N // C)
        def _():
            pltpu.sync_copy(agg_sh.at[pl.ds(k * C, C)],
                            out_hbm.at[pl.ds(cid * N + k * C, C)])
        return 0

    lax.fori_loop(0, ZCH, wb, 0)


_sc_edge = functools.partial(
    pl.kernel,
    out_type=jax.ShapeDtypeStruct((2 * N, DEH), jnp.float32),
    mesh=plsc.VectorSubcoreMesh(core_axis_name="c", subcore_axis_name="s"),
    scratch_types=[
        pltpu.VMEM((C,), jnp.int32),
        pltpu.VMEM((C,), jnp.int32),
        pltpu.VMEM((C, DEH), jnp.float32),
        pltpu.VMEM((C, DEH), jnp.float32),
        pltpu.VMEM((C, DEH), jnp.float32),
        pltpu.VMEM_SHARED((N, DEH), jnp.float32),
        pltpu.SemaphoreType.DMA,
        pltpu.SemaphoreType.DMA,
    ],
)(_sc_edge_body)


def kernel(x, edge_index, edge_attr, u, We, be, Wn, bn):
    src = edge_index[0].astype(jnp.int32)
    dst = edge_index[1].astype(jnp.int32)
    u2 = u.reshape(1, DG)
    be2 = be.reshape(1, DEH)
    bn2 = bn.reshape(1, DF)
    wea = We[:DE]
    ws = We[DE:DE + DF]
    wd = We[DE + DF:DE + 2 * DF]
    weu = We[DE + 2 * DF:]
    wnx = Wn[:DF]
    wna = Wn[DF:DF + DEH]
    wnu = Wn[DF + DEH:]

    xs, xd, ce, cn = pl.pallas_call(
        _pre_body,
        out_shape=[
            jax.ShapeDtypeStruct((N, DEH), jnp.float32),
            jax.ShapeDtypeStruct((N, DEH), jnp.float32),
            jax.ShapeDtypeStruct((1, DEH), jnp.float32),
            jax.ShapeDtypeStruct((1, DF), jnp.float32),
        ],
    )(x, ws, wd, u2, weu, wnu, be2, bn2)

    EB = 8000
    ea = pl.pallas_call(
        _ea_body,
        grid=(E // EB,),
        in_specs=[
            pl.BlockSpec((EB, DE), lambda i: (i, 0)),
            pl.BlockSpec((DE, DEH), lambda i: (0, 0)),
            pl.BlockSpec((1, DEH), lambda i: (0, 0)),
        ],
        out_specs=pl.BlockSpec((EB, DEH), lambda i: (i, 0)),
        out_shape=jax.ShapeDtypeStruct((E, DEH), jnp.float32),
    )(edge_attr, wea, ce)

    agg2 = _sc_edge(ea, src, dst, xs, xd)

    nodes = pl.pallas_call(
        _node_body,
        out_shape=jax.ShapeDtypeStruct((N, DF), jnp.float32),
    )(x, agg2, wnx, wna, cn)
    return nodes
